# baseline (speedup 1.0000x reference)
import functools

import jax
import jax.numpy as jnp
from jax import lax
from jax.experimental import pallas as pl
from jax.experimental.pallas import tpu as pltpu
from jax.experimental.pallas import tpu_sc as plsc

NC = 2
NS = 16
L = 16
NW = NC * NS
BPW = 160
HALF = BPW // 2
KPAD = NW * BPW


def _make_gather(n_rows16):
    mesh = plsc.VectorSubcoreMesh(core_axis_name="c", subcore_axis_name="s")

    @functools.partial(
        pl.kernel,
        mesh=mesh,
        out_type=jax.ShapeDtypeStruct((KPAD * 4,), jnp.float32),
        compiler_params=pltpu.CompilerParams(
            needs_layout_passes=False, use_tc_tiling_on_sc=False
        ),
        scratch_types=[
            pltpu.VMEM((BPW * 3,), jnp.int32),
            pltpu.VMEM((BPW,), jnp.int32),
            pltpu.VMEM((HALF,), jnp.int32),
            pltpu.VMEM((HALF,), jnp.int32),
            pltpu.VMEM((HALF, 16), jnp.float32),
            pltpu.VMEM((HALF, 16), jnp.float32),
            pltpu.VMEM((BPW * 4,), jnp.float32),
            pltpu.SemaphoreType.DMA,
            pltpu.SemaphoreType.DMA,
        ],
    )
    def gather(sel_hbm, table_hbm, out_hbm, selv, bidx, ra, rb, rowsa, rowsb,
               outv, sema, semb):
        wid = lax.axis_index("s") * NC + lax.axis_index("c")
        base = wid * BPW
        pltpu.sync_copy(sel_hbm.at[pl.ds(base * 3, BPW * 3)], selv)
        lane = lax.iota(jnp.int32, 16)
        for j in range(BPW // L):
            pos = lane * 3 + (j * 3 * L + 2)
            box16 = plsc.load_gather(selv, [pos])
            bidx[pl.ds(j * L, L)] = box16
            row16 = box16 >> 2
            if j < HALF // L:
                ra[pl.ds(j * L, L)] = row16
            else:
                rb[pl.ds(j * L - HALF, L)] = row16
        cp_a = pltpu.async_copy(table_hbm.at[ra], rowsa, sema)
        cp_b = pltpu.async_copy(table_hbm.at[rb], rowsb, semb)
        q = lane >> 2
        r = lane & 3
        cp_a.wait()
        for h, rows in ((0, rowsa), (1, rowsb)):
            if h == 1:
                cp_b.wait()
            for g in range(HALF * 4 // L):
                i = q + g * 4
                box16 = plsc.load_gather(bidx, [i + h * HALF])
                vals = plsc.load_gather(rows, [i, (box16 & 3) * 4 + r])
                outv[pl.ds(h * HALF * 4 + g * L, L)] = vals
        pltpu.sync_copy(outv, out_hbm.at[pl.ds(base * 4, BPW * 4)])

    return gather


def kernel(selected_indices, xyxy_boxes):
    k = selected_indices.shape[0]
    sel = selected_indices.astype(jnp.int32)
    sel = jnp.pad(sel, ((0, KPAD - k), (0, 0)))
    sel_flat = sel.reshape(-1)
    table16 = xyxy_boxes.reshape(-1, 16)
    out = _make_gather(table16.shape[0])(sel_flat, table16)
    return out.reshape(KPAD, 4)[:k]


# speedup vs baseline: 1.0079x; 1.0079x over previous
import functools

import jax
import jax.numpy as jnp
from jax import lax
from jax.experimental import pallas as pl
from jax.experimental.pallas import tpu as pltpu
from jax.experimental.pallas import tpu_sc as plsc

NC = 2
NS = 16
L = 16
NW = NC * NS
BPW = 160
HALF = BPW // 2
TAIL_G = 3


def _make_gather(k, n_rows16):
    tail = k - (NW - 1) * BPW
    assert 0 < tail <= BPW and tail % 8 == 0
    mesh = plsc.VectorSubcoreMesh(core_axis_name="c", subcore_axis_name="s")

    @functools.partial(
        pl.kernel,
        mesh=mesh,
        out_type=jax.ShapeDtypeStruct((k * 4,), jnp.float32),
        compiler_params=pltpu.CompilerParams(
            needs_layout_passes=False, use_tc_tiling_on_sc=False
        ),
        scratch_types=[
            pltpu.VMEM((BPW * 3,), jnp.int32),
            pltpu.VMEM((BPW,), jnp.int32),
            pltpu.VMEM((HALF,), jnp.int32),
            pltpu.VMEM((HALF,), jnp.int32),
            pltpu.VMEM((HALF, 16), jnp.float32),
            pltpu.VMEM((HALF, 16), jnp.float32),
            pltpu.VMEM((BPW * 4,), jnp.float32),
            pltpu.SemaphoreType.DMA,
            pltpu.SemaphoreType.DMA,
        ],
    )
    def gather(sel_hbm, table_hbm, out_hbm, selv, bidx, ra, rb, rowsa, rowsb,
               outv, sema, semb):
        wid = lax.axis_index("s") * NC + lax.axis_index("c")
        base = wid * BPW
        lane = lax.iota(jnp.int32, 16)
        q = lane >> 2
        r = lane & 3

        @pl.when(wid < NW - 1)
        def _full():
            pltpu.sync_copy(sel_hbm.at[pl.ds(base * 3, BPW * 3)], selv)
            for j in range(BPW // L):
                pos = lane * 3 + (j * 3 * L + 2)
                box16 = plsc.load_gather(selv, [pos])
                bidx[pl.ds(j * L, L)] = box16
                row16 = box16 >> 2
                if j < HALF // L:
                    ra[pl.ds(j * L, L)] = row16
                else:
                    rb[pl.ds(j * L - HALF, L)] = row16
            cp_a = pltpu.async_copy(table_hbm.at[ra], rowsa, sema)
            cp_b = pltpu.async_copy(table_hbm.at[rb], rowsb, semb)
            cp_a.wait()
            for h, rows in ((0, rowsa), (1, rowsb)):
                if h == 1:
                    cp_b.wait()
                for g in range(HALF * 4 // L):
                    i = q + g * 4
                    box16 = plsc.load_gather(bidx, [i + h * HALF])
                    vals = plsc.load_gather(rows, [i, (box16 & 3) * 4 + r])
                    outv[pl.ds(h * HALF * 4 + g * L, L)] = vals
            pltpu.sync_copy(outv, out_hbm.at[pl.ds(base * 4, BPW * 4)])

        @pl.when(wid == NW - 1)
        def _short_tail():
            pltpu.sync_copy(sel_hbm.at[pl.ds(base * 3, tail * 3)],
                            selv.at[pl.ds(0, tail * 3)])
            for j in range(TAIL_G):
                pos = lane * 3 + (j * 3 * L + 2)
                box16 = plsc.load_gather(selv, [pos])
                box16 = jnp.minimum(jnp.maximum(box16, 0), n_rows16 * 4 - 1)
                bidx[pl.ds(j * L, L)] = box16
                ra[pl.ds(j * L, L)] = box16 >> 2
            pltpu.async_copy(
                table_hbm.at[ra.at[pl.ds(0, TAIL_G * L)]],
                rowsa.at[pl.ds(0, TAIL_G * L)], sema).wait()
            for g in range(tail * 4 // L):
                i = q + g * 4
                box16 = plsc.load_gather(bidx, [i])
                vals = plsc.load_gather(rowsa, [i, (box16 & 3) * 4 + r])
                outv[pl.ds(g * L, L)] = vals
            pltpu.sync_copy(outv.at[pl.ds(0, tail * 4)],
                            out_hbm.at[pl.ds(base * 4, tail * 4)])

    return gather


def kernel(selected_indices, xyxy_boxes):
    k = selected_indices.shape[0]
    sel_flat = selected_indices.astype(jnp.int32).reshape(-1)
    table16 = xyxy_boxes.reshape(-1, 16)
    out = _make_gather(k, table16.shape[0])(sel_flat, table16)
    return out.reshape(k, 4)


# speedup vs baseline: 1.0089x; 1.0010x over previous
import functools

import jax
import jax.numpy as jnp
from jax import lax
from jax.experimental import pallas as pl
from jax.experimental.pallas import tpu as pltpu
from jax.experimental.pallas import tpu_sc as plsc

NC = 2
NS = 16
L = 16
NW = NC * NS
BPW = 160
HALF = BPW // 2
TAIL_G = 3


def _make_gather(k, n_rows16):
    tail = k - (NW - 1) * BPW
    assert 0 < tail <= BPW and tail % 8 == 0
    mesh = plsc.VectorSubcoreMesh(core_axis_name="c", subcore_axis_name="s")

    @functools.partial(
        pl.kernel,
        mesh=mesh,
        out_type=jax.ShapeDtypeStruct((k * 4,), jnp.float32),
        compiler_params=pltpu.CompilerParams(
            needs_layout_passes=False, use_tc_tiling_on_sc=False,
            disable_bounds_checks=True, disable_semaphore_checks=True,
        ),
        scratch_types=[
            pltpu.VMEM((BPW * 3,), jnp.int32),
            pltpu.VMEM((BPW,), jnp.int32),
            pltpu.VMEM((HALF,), jnp.int32),
            pltpu.VMEM((HALF,), jnp.int32),
            pltpu.VMEM((HALF, 16), jnp.float32),
            pltpu.VMEM((HALF, 16), jnp.float32),
            pltpu.VMEM((BPW * 4,), jnp.float32),
            pltpu.SemaphoreType.DMA,
            pltpu.SemaphoreType.DMA,
        ],
    )
    def gather(sel_hbm, table_hbm, out_hbm, selv, bidx, ra, rb, rowsa, rowsb,
               outv, sema, semb):
        wid = lax.axis_index("s") * NC + lax.axis_index("c")
        base = wid * BPW
        lane = lax.iota(jnp.int32, 16)
        q = lane >> 2
        r = lane & 3

        @pl.when(wid < NW - 1)
        def _full():
            pltpu.sync_copy(sel_hbm.at[pl.ds(base * 3, BPW * 3)], selv)
            for j in range(BPW // L):
                pos = lane * 3 + (j * 3 * L + 2)
                box16 = plsc.load_gather(selv, [pos])
                bidx[pl.ds(j * L, L)] = box16
                row16 = box16 >> 2
                if j < HALF // L:
                    ra[pl.ds(j * L, L)] = row16
                else:
                    rb[pl.ds(j * L - HALF, L)] = row16
            cp_a = pltpu.async_copy(table_hbm.at[ra], rowsa, sema)
            cp_b = pltpu.async_copy(table_hbm.at[rb], rowsb, semb)
            cp_a.wait()
            for h, rows in ((0, rowsa), (1, rowsb)):
                if h == 1:
                    cp_b.wait()
                for g in range(HALF * 4 // L):
                    i = q + g * 4
                    box16 = plsc.load_gather(bidx, [i + h * HALF])
                    vals = plsc.load_gather(rows, [i, (box16 & 3) * 4 + r])
                    outv[pl.ds(h * HALF * 4 + g * L, L)] = vals
            pltpu.sync_copy(outv, out_hbm.at[pl.ds(base * 4, BPW * 4)])

        @pl.when(wid == NW - 1)
        def _short_tail():
            pltpu.sync_copy(sel_hbm.at[pl.ds(base * 3, tail * 3)],
                            selv.at[pl.ds(0, tail * 3)])
            for j in range(TAIL_G):
                pos = lane * 3 + (j * 3 * L + 2)
                box16 = plsc.load_gather(selv, [pos])
                box16 = jnp.minimum(jnp.maximum(box16, 0), n_rows16 * 4 - 1)
                bidx[pl.ds(j * L, L)] = box16
                ra[pl.ds(j * L, L)] = box16 >> 2
            pltpu.async_copy(
                table_hbm.at[ra.at[pl.ds(0, TAIL_G * L)]],
                rowsa.at[pl.ds(0, TAIL_G * L)], sema).wait()
            for g in range(tail * 4 // L):
                i = q + g * 4
                box16 = plsc.load_gather(bidx, [i])
                vals = plsc.load_gather(rowsa, [i, (box16 & 3) * 4 + r])
                outv[pl.ds(g * L, L)] = vals
            pltpu.sync_copy(outv.at[pl.ds(0, tail * 4)],
                            out_hbm.at[pl.ds(base * 4, tail * 4)])

    return gather


def kernel(selected_indices, xyxy_boxes):
    k = selected_indices.shape[0]
    sel_flat = selected_indices.astype(jnp.int32).reshape(-1)
    table16 = xyxy_boxes.reshape(-1, 16)
    out = _make_gather(k, table16.shape[0])(sel_flat, table16)
    return out.reshape(k, 4)


# speedup vs baseline: 1.8697x; 1.8532x over previous
import functools

import jax
import jax.numpy as jnp
from jax import lax
from jax.experimental import pallas as pl
from jax.experimental.pallas import tpu as pltpu
from jax.experimental.pallas import tpu_sc as plsc

NC = 2
NS = 16
L = 16
NW = NC * NS
BPW = 160
NSTREAM = BPW * 4 // 128
TAIL_G = 3


def _make_gather(k, n):
    tail = k - (NW - 1) * BPW
    assert 0 < tail <= BPW and tail % 8 == 0 and (BPW * 4) % 128 == 0
    tg16 = TAIL_G * L
    mesh = plsc.VectorSubcoreMesh(core_axis_name="c", subcore_axis_name="s")

    @functools.partial(
        pl.kernel,
        mesh=mesh,
        out_type=jax.ShapeDtypeStruct((4 * k,), jnp.float32),
        compiler_params=pltpu.CompilerParams(
            needs_layout_passes=False, use_tc_tiling_on_sc=False,
            disable_bounds_checks=True, disable_semaphore_checks=True,
        ),
        scratch_types=[
            pltpu.VMEM((BPW,), jnp.int32),
            pltpu.VMEM((BPW * 4,), jnp.int32),
            pltpu.VMEM((BPW * 4,), jnp.float32),
            pltpu.SemaphoreType.DMA,
        ],
    )
    def gather(idx_hbm, table_hbm, out_hbm, bidx, widx, vals, sem):
        wid = lax.axis_index("s") * NC + lax.axis_index("c")
        base = wid * BPW

        @pl.when(wid < NW - 1)
        def _full():
            pltpu.sync_copy(idx_hbm.at[pl.ds(base, BPW)], bidx)
            for g in range(BPW // L):
                b16 = bidx[pl.ds(g * L, L)]
                for c in range(4):
                    widx[pl.ds(c * BPW + g * L, L)] = b16 + c * n
            cps = [
                pltpu.async_copy(
                    table_hbm.at[widx.at[pl.ds(s * 128, 128)]],
                    vals.at[pl.ds(s * 128, 128)], sem)
                for s in range(NSTREAM)
            ]
            for cp in cps:
                cp.wait()
            for c in range(4):
                pltpu.sync_copy(vals.at[pl.ds(c * BPW, BPW)],
                                out_hbm.at[pl.ds(c * k + base, BPW)])

        @pl.when(wid == NW - 1)
        def _short_tail():
            pltpu.sync_copy(idx_hbm.at[pl.ds(base, tail)],
                            bidx.at[pl.ds(0, tail)])
            for g in range(TAIL_G):
                b16 = bidx[pl.ds(g * L, L)]
                b16 = jnp.minimum(jnp.maximum(b16, 0), n - 1)
                for c in range(4):
                    widx[pl.ds(c * tg16 + g * L, L)] = b16 + c * n
            cps = [
                pltpu.async_copy(
                    table_hbm.at[widx.at[pl.ds(s * 64, 64)]],
                    vals.at[pl.ds(s * 64, 64)], sem)
                for s in range(4 * tg16 // 64)
            ]
            for cp in cps:
                cp.wait()
            for c in range(4):
                pltpu.sync_copy(vals.at[pl.ds(c * tg16, tail)],
                                out_hbm.at[pl.ds(c * k + base, tail)])

    return gather


def kernel(selected_indices, xyxy_boxes):
    k = selected_indices.shape[0]
    n = xyxy_boxes.shape[1]
    box_idx = selected_indices[:, 2].astype(jnp.int32)
    table_cm = xyxy_boxes[0].T.reshape(-1)
    out = _make_gather(k, n)(box_idx, table_cm)
    return out.reshape(4, k).T
